# Initial kernel scaffold; baseline (speedup 1.0000x reference)
import jax
import jax.numpy as jnp
from jax.experimental import pallas as pl


def kernel(x, edge_index, W_cheb, b_cheb, W_lin, b_lin):
    raise NotImplementedError("write your pallas kernel here")


# speedup vs baseline: 4.8421x; 4.8421x over previous
import functools

import jax
import jax.numpy as jnp
from jax import lax
from jax.experimental import pallas as pl
from jax.experimental.pallas import tpu as pltpu
from jax.experimental.pallas import tpu_sc as plsc

B = 4
N = 10000
E = 320000
DIN = 128
DOUT = 128

NPAD = 10240
ROWS_PER_TILE = NPAD // 16
CHUNK = 128
EPT = 20480
NCHUNKS = EPT // CHUNK
E_PAD = EPT * 16
DEG_CHUNKS = E_PAD // 32 // CHUNK

@functools.cache
def _mesh():
    return plsc.VectorSubcoreMesh(core_axis_name="c", subcore_axis_name="s")


def _sc_deg_body(dsts, ones_pat, zeros_slab, deg_out, dst_v, ones_v, deg_sh,
                 sem):
    c = lax.axis_index("c")
    s = lax.axis_index("s")
    pltpu.sync_copy(dsts.at[c].at[s], dst_v)
    pltpu.sync_copy(ones_pat, ones_v)
    pltpu.sync_copy(
        zeros_slab, deg_sh.at[pl.ds(s * ROWS_PER_TILE, ROWS_PER_TILE)])
    plsc.subcore_barrier()

    @pl.loop(0, DEG_CHUNKS)
    def _(j):
        pltpu.sync_copy(ones_v, deg_sh.at[dst_v.at[j]], add=True)

    plsc.subcore_barrier()
    pltpu.sync_copy(
        deg_sh.at[pl.ds(s * ROWS_PER_TILE, ROWS_PER_TILE)],
        deg_out.at[c].at[pl.ds(s * ROWS_PER_TILE, ROWS_PER_TILE)],
    )


@jax.jit
def _sc_deg(dsts32, ones_pat, zeros_slab):
    return pl.kernel(
        _sc_deg_body,
        out_type=jax.ShapeDtypeStruct((2, NPAD, DIN), jnp.float32),
        mesh=_mesh(),
        scratch_types=[
            pltpu.VMEM((DEG_CHUNKS, CHUNK), jnp.int32),
            pltpu.VMEM((CHUNK, DIN), jnp.float32),
            pltpu.VMEM_SHARED((NPAD, DIN), jnp.float32),
            pltpu.SemaphoreType.DMA,
        ],
    )(dsts32, ones_pat, zeros_slab)


def _lap_batch(table, edges, agg_out, b, s, i0, i1, buf0, buf1, agg_sh,
               semi0, semi1, semg0, semg1):
    row0 = s * ROWS_PER_TILE
    tab = table.at[b]
    eds = edges.at[s]
    half = NCHUNKS // 2

    pltpu.async_copy(eds.at[0], i0, semi0)
    pltpu.async_copy(eds.at[1], i1, semi1)
    pltpu.make_async_copy(eds.at[0], i0, semi0).wait()
    pltpu.async_copy(tab.at[i0.at[0]], buf0, semg0)

    @pl.loop(0, half)
    def _(i):
        j0 = 2 * i
        pltpu.make_async_copy(eds.at[0], i1, semi1).wait()
        pltpu.async_copy(tab.at[i1.at[0]], buf1, semg1)
        pltpu.make_async_copy(tab.at[i0.at[0]], buf0, semg0).wait()
        pltpu.sync_copy(buf0, agg_sh.at[i0.at[1]], add=True)

        @pl.when(i < half - 1)
        def _():
            pltpu.async_copy(eds.at[j0 + 2], i0, semi0)
            pltpu.make_async_copy(eds.at[0], i0, semi0).wait()
            pltpu.async_copy(tab.at[i0.at[0]], buf0, semg0)

        pltpu.make_async_copy(tab.at[i1.at[0]], buf1, semg1).wait()
        pltpu.sync_copy(buf1, agg_sh.at[i1.at[1]], add=True)

        @pl.when(i < half - 1)
        def _():
            pltpu.async_copy(eds.at[j0 + 3], i1, semi1)

    plsc.subcore_barrier()
    pltpu.sync_copy(
        agg_sh.at[pl.ds(row0, ROWS_PER_TILE)],
        agg_out.at[b].at[pl.ds(row0, ROWS_PER_TILE)],
    )
    plsc.subcore_barrier()


def _lap_zero(zeros_slab, agg_sh, s):
    pltpu.sync_copy(
        zeros_slab, agg_sh.at[pl.ds(s * ROWS_PER_TILE, ROWS_PER_TILE)])
    plsc.subcore_barrier()


def _sc_lap_body(table, edges, zeros_slab, agg_out, i0, i1, buf0, buf1,
                 agg_sh, semi0, semi1, semg0, semg1):
    c = lax.axis_index("c")
    s = lax.axis_index("s")

    @pl.when(c == 0)
    def _():
        for b in (0, 1):
            _lap_zero(zeros_slab, agg_sh, s)
            _lap_batch(table, edges, agg_out, b, s, i0, i1, buf0, buf1,
                       agg_sh, semi0, semi1, semg0, semg1)

    @pl.when(c == 1)
    def _():
        for b in (2, 3):
            _lap_zero(zeros_slab, agg_sh, s)
            _lap_batch(table, edges, agg_out, b, s, i0, i1, buf0, buf1,
                       agg_sh, semi0, semi1, semg0, semg1)


@jax.jit
def _sc_lap(table, edges, zeros_slab):
    return pl.kernel(
        _sc_lap_body,
        out_type=jax.ShapeDtypeStruct((B, NPAD, DIN), jnp.float32),
        mesh=_mesh(),
        scratch_types=[
            pltpu.VMEM((2, CHUNK), jnp.int32),
            pltpu.VMEM((2, CHUNK), jnp.int32),
            pltpu.VMEM((CHUNK, DIN), jnp.float32),
            pltpu.VMEM((CHUNK, DIN), jnp.float32),
            pltpu.VMEM_SHARED((NPAD, DIN), jnp.float32),
            pltpu.SemaphoreType.DMA,
            pltpu.SemaphoreType.DMA,
            pltpu.SemaphoreType.DMA,
            pltpu.SemaphoreType.DMA,
        ],
    )(table, edges, zeros_slab)


_BLK = 1000


def _tc_prescale_body(x_ref, dp_ref, u0_ref, dis_ref):
    deg = dp_ref[0, :, 0] + dp_ref[1, :, 0]
    dis = jnp.where(deg > 0, lax.rsqrt(jnp.maximum(deg, 1.0)), 0.0)
    u0_ref[0] = x_ref[0] * dis[:, None]
    dis_ref[...] = jnp.broadcast_to(dis[:, None], (_BLK, 8))


@jax.jit
def _tc_prescale(x, deg_parts):
    return pl.pallas_call(
        _tc_prescale_body,
        grid=(B, N // _BLK),
        in_specs=[
            pl.BlockSpec((1, _BLK, DIN), lambda b, i: (b, i, 0)),
            pl.BlockSpec((2, _BLK, DIN), lambda b, i: (0, i, 0)),
        ],
        out_specs=[
            pl.BlockSpec((1, _BLK, DIN), lambda b, i: (b, i, 0)),
            pl.BlockSpec((_BLK, 8), lambda b, i: (i, 0)),
        ],
        out_shape=[
            jax.ShapeDtypeStruct((B, N, DIN), jnp.float32),
            jax.ShapeDtypeStruct((N, 8), jnp.float32),
        ],
    )(x, deg_parts)


def _tc_mid_body(agg_ref, dis_ref, u1_ref):
    dis = dis_ref[:, 0:1]
    u1_ref[0] = (-dis * dis) * agg_ref[0]


@jax.jit
def _tc_mid(agg1, dis):
    return pl.pallas_call(
        _tc_mid_body,
        grid=(B, N // _BLK),
        in_specs=[
            pl.BlockSpec((1, _BLK, DIN), lambda b, i: (b, i, 0)),
            pl.BlockSpec((_BLK, 8), lambda b, i: (i, 0)),
        ],
        out_specs=pl.BlockSpec((1, _BLK, DIN), lambda b, i: (b, i, 0)),
        out_shape=jax.ShapeDtypeStruct((B, N, DIN), jnp.float32),
    )(agg1, dis)


def _tc_final_body(x_ref, a1_ref, a2_ref, dis_ref, wc_ref, bc_ref, wl_ref,
                   bl_ref, out_ref):
    dis = dis_ref[:, 0:1]
    t0 = x_ref[0]
    t1 = -dis * a1_ref[0]
    t2 = (-2.0 * dis) * a2_ref[0] - t0
    wc = wc_ref[...]
    h = jnp.dot(t0, wc[0:DIN], preferred_element_type=jnp.float32)
    h += jnp.dot(t1, wc[DIN:2 * DIN], preferred_element_type=jnp.float32)
    h += jnp.dot(t2, wc[2 * DIN:3 * DIN], preferred_element_type=jnp.float32)
    h += bc_ref[...]
    h = jnp.maximum(h, 0.0)
    out = jnp.dot(h, wl_ref[...], preferred_element_type=jnp.float32)
    out_ref[0] = out + bl_ref[...]


@jax.jit
def _tc_final(x, agg1, agg2, dis, W_cheb, b_cheb, W_lin, b_lin):
    return pl.pallas_call(
        _tc_final_body,
        grid=(B, N // _BLK),
        in_specs=[
            pl.BlockSpec((1, _BLK, DIN), lambda b, i: (b, i, 0)),
            pl.BlockSpec((1, _BLK, DIN), lambda b, i: (b, i, 0)),
            pl.BlockSpec((1, _BLK, DIN), lambda b, i: (b, i, 0)),
            pl.BlockSpec((_BLK, 8), lambda b, i: (i, 0)),
            pl.BlockSpec((3 * DIN, DOUT), lambda b, i: (0, 0)),
            pl.BlockSpec((1, DOUT), lambda b, i: (0, 0)),
            pl.BlockSpec((DOUT, DIN), lambda b, i: (0, 0)),
            pl.BlockSpec((1, DIN), lambda b, i: (0, 0)),
        ],
        out_specs=pl.BlockSpec((1, _BLK, DIN), lambda b, i: (b, i, 0)),
        out_shape=jax.ShapeDtypeStruct((B, N, DIN), jnp.float32),
    )(x, agg1, agg2, dis, W_cheb, b_cheb.reshape(1, DOUT), W_lin,
      b_lin.reshape(1, DIN))


@jax.jit
def kernel(x, edge_index, W_cheb, b_cheb, W_lin, b_lin):
    src = edge_index[0]
    dst = edge_index[1]
    pad = E_PAD - E
    src_p = jnp.concatenate([src, jnp.zeros((pad,), jnp.int32)])
    dst_p = jnp.concatenate([dst, jnp.full((pad,), N, jnp.int32)])
    srcs = src_p.reshape(16, NCHUNKS, 1, CHUNK)
    dsts = dst_p.reshape(16, NCHUNKS, 1, CHUNK)
    edges = jnp.concatenate([srcs, dsts], axis=2)
    dsts32 = dst_p.reshape(2, 16, DEG_CHUNKS, CHUNK)

    ones_pat = jnp.ones((CHUNK, DIN), jnp.float32)
    zeros_slab = jnp.zeros((ROWS_PER_TILE, DIN), jnp.float32)

    deg_parts = _sc_deg(dsts32, ones_pat, zeros_slab)
    u0, dis = _tc_prescale(x, deg_parts)
    agg1 = _sc_lap(u0, edges, zeros_slab)
    u1 = _tc_mid(agg1[:, :N], dis)
    agg2 = _sc_lap(u1, edges, zeros_slab)
    return _tc_final(x, agg1[:, :N], agg2[:, :N], dis, W_cheb, b_cheb,
                     W_lin, b_lin)
